# baseline (speedup 1.0000x reference)
import functools

import jax
import jax.numpy as jnp
import numpy as np
from jax import lax
from jax.experimental import pallas as pl
from jax.experimental.pallas import tpu as pltpu
from jax.experimental.pallas import tpu_sc as plsc

INF = 100000000.0
L = 20000
T = 100
NCORES = 2
NSUB = 16
LANES = 16
NW = NCORES * NSUB
NCOEF = 13

NSC = 6144
PER_TILE = NSC // NW
NVREG = PER_TILE // LANES
KUN = 4
NGRP = NVREG // KUN

LTC = L - NSC
TCROWS = -(-LTC // 1024)
LTCPAD = TCROWS * 1024


def _build_coeffs(corners):
    c = corners.reshape(T, 4, 2)
    xi = c[:, :, 0]
    yi = c[:, :, 1]
    F = jnp.roll(yi, -1, axis=1) - yi
    G = jnp.roll(xi, -1, axis=1) - xi
    inv = 1.0 / (jnp.sqrt(F * F + G * G) + 1e-12)

    def tri(a, b, cc):
        u = a - cc
        v = b - cc
        return 0.5 * jnp.abs(u[..., 0] * v[..., 1] - u[..., 1] * v[..., 0])

    c0, c1, c2, c3 = c[:, 0], c[:, 1], c[:, 2], c[:, 3]
    poly = tri(c0, c1, c2) + tri(c2, c3, c0)
    thr2 = 2.0 * (poly + 0.001)
    return jnp.concatenate([xi, yi, inv, thr2[:, None]], axis=1)


def _sqrt16(x):
    i = lax.bitcast_convert_type(x, jnp.int32)
    i = np.int32(0x5F3759DF) - lax.shift_right_arithmetic(i, 1)
    y = lax.bitcast_convert_type(i, jnp.float32)
    for _ in range(3):
        y = y * (1.5 - 0.5 * x * y * y)
    return jnp.where(x > 0, x * y, jnp.zeros_like(x))


@functools.partial(
    pl.kernel,
    out_type=jax.ShapeDtypeStruct((NSC,), jnp.float32),
    mesh=plsc.VectorSubcoreMesh(core_axis_name="c", subcore_axis_name="s"),
    scratch_types=[
        pltpu.VMEM((T * NCOEF * LANES,), jnp.float32),
        pltpu.VMEM((PER_TILE,), jnp.float32),
        pltpu.VMEM((PER_TILE,), jnp.float32),
        pltpu.VMEM((PER_TILE,), jnp.float32),
    ],
)
def _sc_ctr(coef_hbm, xs_hbm, ys_hbm, out_hbm, coef_v, xs_v, ys_v, out_v):
    wid = lax.axis_index("s") * NCORES + lax.axis_index("c")
    base = wid * PER_TILE
    pltpu.sync_copy(coef_hbm, coef_v)
    pltpu.sync_copy(xs_hbm.at[pl.ds(base, PER_TILE)], xs_v)
    pltpu.sync_copy(ys_hbm.at[pl.ds(base, PER_TILE)], ys_v)

    big = jnp.full((LANES,), 2.0 * INF, jnp.float32)
    zero = jnp.zeros((LANES,), jnp.float32)
    inf_v = jnp.full((LANES,), INF, jnp.float32)

    for g in range(NGRP):
        xk = [xs_v[pl.ds((g * KUN + k) * LANES, LANES)] for k in range(KUN)]
        yk = [ys_v[pl.ds((g * KUN + k) * LANES, LANES)] for k in range(KUN)]
        init = tuple([big, zero, zero, zero, zero] * KUN)

        TUN = 1

        def body(tt, carry, xk=xk, yk=yk):
            out = list(carry)
            for u in range(TUN):
                row = (tt * TUN + u) * (NCOEF * LANES)
                cxi = [coef_v[pl.ds(row + e * LANES, LANES)] for e in range(4)]
                cyi = [
                    coef_v[pl.ds(row + (4 + e) * LANES, LANES)] for e in range(4)
                ]
                cinv = [
                    coef_v[pl.ds(row + (8 + e) * LANES, LANES)] for e in range(4)
                ]
                thr2 = coef_v[pl.ds(row + 12 * LANES, LANES)]
                for k in range(KUN):
                    best, d0, d1, d2, d3 = out[5 * k : 5 * k + 5]
                    dx = [cxi[e] - xk[k] for e in range(4)]
                    dy = [cyi[e] - yk[k] for e in range(4)]
                    w = [
                        jnp.abs(dx[e] * dy[(e + 1) % 4] - dy[e] * dx[(e + 1) % 4])
                        for e in range(4)
                    ]
                    s = ((w[0] + w[1]) + w[2]) + w[3]
                    d = [w[e] * cinv[e] for e in range(4)]
                    maxd = jnp.maximum(
                        jnp.maximum(d[0], d[2]), jnp.maximum(d[1], d[3])
                    )
                    masked = jnp.where(s <= thr2, maxd, inf_v)
                    upd = masked < best
                    out[5 * k : 5 * k + 5] = [
                        jnp.where(upd, masked, best),
                        jnp.where(upd, d[0], d0),
                        jnp.where(upd, d[1], d1),
                        jnp.where(upd, d[2], d2),
                        jnp.where(upd, d[3], d3),
                    ]
            return tuple(out)

        res = lax.fori_loop(0, T // TUN, body, init)
        for k in range(KUN):
            _, d0, d1, d2, d3 = res[5 * k : 5 * k + 5]
            i02 = jnp.minimum(d0, d2)
            a02 = jnp.maximum(d0, d2)
            i13 = jnp.minimum(d1, d3)
            a13 = jnp.maximum(d1, d3)
            r1 = i02 / jnp.maximum(a02, 1e-12)
            r2 = i13 / jnp.maximum(a13, 1e-12)
            out_v[pl.ds((g * KUN + k) * LANES, LANES)] = _sqrt16(r1 * r2)

    pltpu.sync_copy(out_v, out_hbm.at[pl.ds(base, PER_TILE)])


def _tc_body(coef_smem, xs_ref, ys_ref, out_ref):
    inf_v = jnp.full((8, 128), INF, jnp.float32)
    big = jnp.full((8, 128), 2.0 * INF, jnp.float32)
    zero = jnp.zeros((8, 128), jnp.float32)
    ROWGRP = 3
    chunks = [
        list(range(c, min(c + ROWGRP, TCROWS))) for c in range(0, TCROWS, ROWGRP)
    ]
    for rows in chunks:
        xk = [xs_ref[pl.ds(8 * r, 8), :] for r in rows]
        yk = [ys_ref[pl.ds(8 * r, 8), :] for r in rows]
        init = tuple([big, zero, zero, zero, zero] * len(rows))

        def body(t, carry, xk=xk, yk=yk, nr=len(rows)):
            row = t * NCOEF
            cxi = [coef_smem[row + e] for e in range(4)]
            cyi = [coef_smem[row + 4 + e] for e in range(4)]
            cinv = [coef_smem[row + 8 + e] for e in range(4)]
            thr2 = coef_smem[row + 12]
            out = []
            for k in range(nr):
                best, d0, d1, d2, d3 = carry[5 * k : 5 * k + 5]
                dx = [cxi[e] - xk[k] for e in range(4)]
                dy = [cyi[e] - yk[k] for e in range(4)]
                w = [
                    jnp.abs(dx[e] * dy[(e + 1) % 4] - dy[e] * dx[(e + 1) % 4])
                    for e in range(4)
                ]
                s = ((w[0] + w[1]) + w[2]) + w[3]
                d = [w[e] * cinv[e] for e in range(4)]
                maxd = jnp.maximum(
                    jnp.maximum(d[0], d[2]), jnp.maximum(d[1], d[3])
                )
                masked = jnp.where(s <= thr2, maxd, inf_v)
                upd = masked < best
                out += [
                    jnp.where(upd, masked, best),
                    jnp.where(upd, d[0], d0),
                    jnp.where(upd, d[1], d1),
                    jnp.where(upd, d[2], d2),
                    jnp.where(upd, d[3], d3),
                ]
            return tuple(out)

        res = lax.fori_loop(0, T, body, init)
        for k, r in enumerate(rows):
            _, d0, d1, d2, d3 = res[5 * k : 5 * k + 5]
            i02 = jnp.minimum(d0, d2)
            a02 = jnp.maximum(d0, d2)
            i13 = jnp.minimum(d1, d3)
            a13 = jnp.maximum(d1, d3)
            r1 = i02 / jnp.maximum(a02, 1e-12)
            r2 = i13 / jnp.maximum(a13, 1e-12)
            out_ref[pl.ds(8 * r, 8), :] = jnp.sqrt(r1 * r2)


_tc_ctr = pl.pallas_call(
    _tc_body,
    out_shape=jax.ShapeDtypeStruct((TCROWS * 8, 128), jnp.float32),
    in_specs=[
        pl.BlockSpec(memory_space=pltpu.SMEM),
        pl.BlockSpec(memory_space=pltpu.VMEM),
        pl.BlockSpec(memory_space=pltpu.VMEM),
    ],
    out_specs=pl.BlockSpec(memory_space=pltpu.VMEM),
)


def kernel(corners, xs, ys):
    coef = _build_coeffs(corners)
    coef_rep = jnp.broadcast_to(
        coef.reshape(T * NCOEF, 1), (T * NCOEF, LANES)
    ).reshape(T * NCOEF * LANES)
    coef_flat = coef.reshape(T * NCOEF)
    xs_sc, ys_sc = xs[:NSC], ys[:NSC]
    xs_tc = jnp.pad(xs[NSC:], (0, LTCPAD - LTC)).reshape(TCROWS * 8, 128)
    ys_tc = jnp.pad(ys[NSC:], (0, LTCPAD - LTC)).reshape(TCROWS * 8, 128)
    out_sc = _sc_ctr(coef_rep, xs_sc, ys_sc)
    out_tc = _tc_ctr(coef_flat, xs_tc, ys_tc)
    return jnp.concatenate([out_sc, out_tc.reshape(LTCPAD)[:LTC]])


# speedup vs baseline: 1.0082x; 1.0082x over previous
import functools

import jax
import jax.numpy as jnp
import numpy as np
from jax import lax
from jax.experimental import pallas as pl
from jax.experimental.pallas import tpu as pltpu
from jax.experimental.pallas import tpu_sc as plsc

INF = 100000000.0
L = 20000
T = 100
NCORES = 2
NSUB = 16
LANES = 16
NW = NCORES * NSUB
NCOEF = 13

NSC = 6144
PER_TILE = NSC // NW
NVREG = PER_TILE // LANES
KUN = 4
NGRP = NVREG // KUN

LTC = L - NSC
TCROWS = -(-LTC // 1024)
LTCPAD = TCROWS * 1024


def _build_coeffs(corners):
    c = corners.reshape(T, 4, 2)
    xi = c[:, :, 0]
    yi = c[:, :, 1]
    F = jnp.roll(yi, -1, axis=1) - yi
    G = jnp.roll(xi, -1, axis=1) - xi
    inv = 1.0 / (jnp.sqrt(F * F + G * G) + 1e-12)

    def tri(a, b, cc):
        u = a - cc
        v = b - cc
        return 0.5 * jnp.abs(u[..., 0] * v[..., 1] - u[..., 1] * v[..., 0])

    c0, c1, c2, c3 = c[:, 0], c[:, 1], c[:, 2], c[:, 3]
    poly = tri(c0, c1, c2) + tri(c2, c3, c0)
    thr2 = 2.0 * (poly + 0.001)
    return jnp.concatenate([xi, yi, inv, thr2[:, None]], axis=1)


def _sqrt16(x):
    i = lax.bitcast_convert_type(x, jnp.int32)
    i = np.int32(0x5F3759DF) - lax.shift_right_arithmetic(i, 1)
    y = lax.bitcast_convert_type(i, jnp.float32)
    for _ in range(3):
        y = y * (1.5 - 0.5 * x * y * y)
    return jnp.where(x > 0, x * y, jnp.zeros_like(x))


@functools.partial(
    pl.kernel,
    out_type=jax.ShapeDtypeStruct((NSC,), jnp.float32),
    mesh=plsc.VectorSubcoreMesh(core_axis_name="c", subcore_axis_name="s"),
    scratch_types=[
        pltpu.VMEM((T * NCOEF * LANES,), jnp.float32),
        pltpu.VMEM((PER_TILE,), jnp.float32),
        pltpu.VMEM((PER_TILE,), jnp.float32),
        pltpu.VMEM((PER_TILE,), jnp.float32),
    ],
)
def _sc_ctr(coef_hbm, xs_hbm, ys_hbm, out_hbm, coef_v, xs_v, ys_v, out_v):
    wid = lax.axis_index("s") * NCORES + lax.axis_index("c")
    base = wid * PER_TILE
    pltpu.sync_copy(coef_hbm, coef_v)
    pltpu.sync_copy(xs_hbm.at[pl.ds(base, PER_TILE)], xs_v)
    pltpu.sync_copy(ys_hbm.at[pl.ds(base, PER_TILE)], ys_v)

    big = jnp.full((LANES,), 2.0 * INF, jnp.float32)
    zero = jnp.zeros((LANES,), jnp.float32)
    inf_v = jnp.full((LANES,), INF, jnp.float32)

    for g in range(NGRP):
        xk = [xs_v[pl.ds((g * KUN + k) * LANES, LANES)] for k in range(KUN)]
        yk = [ys_v[pl.ds((g * KUN + k) * LANES, LANES)] for k in range(KUN)]
        init = tuple([big, zero, zero, zero, zero] * KUN)

        TUN = 1

        def body(tt, carry, xk=xk, yk=yk):
            out = list(carry)
            for u in range(TUN):
                row = (tt * TUN + u) * (NCOEF * LANES)
                cxi = [coef_v[pl.ds(row + e * LANES, LANES)] for e in range(4)]
                cyi = [
                    coef_v[pl.ds(row + (4 + e) * LANES, LANES)] for e in range(4)
                ]
                cinv = [
                    coef_v[pl.ds(row + (8 + e) * LANES, LANES)] for e in range(4)
                ]
                thr2 = coef_v[pl.ds(row + 12 * LANES, LANES)]
                for k in range(KUN):
                    best, d0, d1, d2, d3 = out[5 * k : 5 * k + 5]
                    dx = [cxi[e] - xk[k] for e in range(4)]
                    dy = [cyi[e] - yk[k] for e in range(4)]
                    w = [
                        jnp.abs(dx[e] * dy[(e + 1) % 4] - dy[e] * dx[(e + 1) % 4])
                        for e in range(4)
                    ]
                    s = ((w[0] + w[1]) + w[2]) + w[3]
                    d = [w[e] * cinv[e] for e in range(4)]
                    maxd = jnp.maximum(
                        jnp.maximum(d[0], d[2]), jnp.maximum(d[1], d[3])
                    )
                    masked = jnp.where(s <= thr2, maxd, inf_v)
                    upd = masked < best
                    out[5 * k : 5 * k + 5] = [
                        jnp.where(upd, masked, best),
                        jnp.where(upd, d[0], d0),
                        jnp.where(upd, d[1], d1),
                        jnp.where(upd, d[2], d2),
                        jnp.where(upd, d[3], d3),
                    ]
            return tuple(out)

        res = lax.fori_loop(0, T // TUN, body, init)
        for k in range(KUN):
            _, d0, d1, d2, d3 = res[5 * k : 5 * k + 5]
            i02 = jnp.minimum(d0, d2)
            a02 = jnp.maximum(d0, d2)
            i13 = jnp.minimum(d1, d3)
            a13 = jnp.maximum(d1, d3)
            r1 = i02 / jnp.maximum(a02, 1e-12)
            r2 = i13 / jnp.maximum(a13, 1e-12)
            out_v[pl.ds((g * KUN + k) * LANES, LANES)] = _sqrt16(r1 * r2)

    pltpu.sync_copy(out_v, out_hbm.at[pl.ds(base, PER_TILE)])


def _tc_body(coef_smem, xs_ref, ys_ref, out_ref):
    inf_v = jnp.full((8, 128), INF, jnp.float32)
    big = jnp.full((8, 128), 2.0 * INF, jnp.float32)
    zero = jnp.zeros((8, 128), jnp.float32)
    ROWGRP = 3
    chunks = [
        list(range(c, min(c + ROWGRP, TCROWS))) for c in range(0, TCROWS, ROWGRP)
    ]
    for rows in chunks:
        xk = [xs_ref[pl.ds(8 * r, 8), :] for r in rows]
        yk = [ys_ref[pl.ds(8 * r, 8), :] for r in rows]
        init = tuple([big, zero, zero, zero, zero] * len(rows))

        def body(t, carry, xk=xk, yk=yk, nr=len(rows)):
            row = t * NCOEF
            cxi = [coef_smem[row + e] for e in range(4)]
            cyi = [coef_smem[row + 4 + e] for e in range(4)]
            cinv = [coef_smem[row + 8 + e] for e in range(4)]
            thr2 = coef_smem[row + 12]
            out = []
            for k in range(nr):
                best, d0, d1, d2, d3 = carry[5 * k : 5 * k + 5]
                dx = [cxi[e] - xk[k] for e in range(4)]
                dy = [cyi[e] - yk[k] for e in range(4)]
                w = [
                    jnp.abs(dx[e] * dy[(e + 1) % 4] - dy[e] * dx[(e + 1) % 4])
                    for e in range(4)
                ]
                s = ((w[0] + w[1]) + w[2]) + w[3]
                d = [w[e] * cinv[e] for e in range(4)]
                maxd = jnp.maximum(
                    jnp.maximum(d[0], d[2]), jnp.maximum(d[1], d[3])
                )
                masked = jnp.where(s <= thr2, maxd, inf_v)
                upd = masked < best
                out += [
                    jnp.where(upd, masked, best),
                    jnp.where(upd, d[0], d0),
                    jnp.where(upd, d[1], d1),
                    jnp.where(upd, d[2], d2),
                    jnp.where(upd, d[3], d3),
                ]
            return tuple(out)

        res = lax.fori_loop(0, T, body, init)
        for k, r in enumerate(rows):
            _, d0, d1, d2, d3 = res[5 * k : 5 * k + 5]
            i02 = jnp.minimum(d0, d2)
            a02 = jnp.maximum(d0, d2)
            i13 = jnp.minimum(d1, d3)
            a13 = jnp.maximum(d1, d3)
            r1 = i02 / jnp.maximum(a02, 1e-12)
            r2 = i13 / jnp.maximum(a13, 1e-12)
            out_ref[pl.ds(8 * r, 8), :] = jnp.sqrt(r1 * r2)


_tc_ctr = pl.pallas_call(
    _tc_body,
    out_shape=jax.ShapeDtypeStruct((TCROWS * 8, 128), jnp.float32),
    in_specs=[
        pl.BlockSpec(memory_space=pltpu.SMEM),
        pl.BlockSpec(memory_space=pltpu.VMEM),
        pl.BlockSpec(memory_space=pltpu.VMEM),
    ],
    out_specs=pl.BlockSpec(memory_space=pltpu.VMEM),
)


def kernel(corners, xs, ys):
    coef = _build_coeffs(corners)
    coef_rep = jnp.broadcast_to(
        coef.reshape(T * NCOEF, 1), (T * NCOEF, LANES)
    ).reshape(T * NCOEF * LANES)
    coef_flat = coef.reshape(T * NCOEF)
    xs_sc, ys_sc = xs[:NSC], ys[:NSC]
    xs_tc = jnp.pad(xs[NSC:], (0, LTCPAD - LTC)).reshape(TCROWS * 8, 128)
    ys_tc = jnp.pad(ys[NSC:], (0, LTCPAD - LTC)).reshape(TCROWS * 8, 128)
    out_tc = _tc_ctr(coef_flat, xs_tc, ys_tc)
    out_sc = _sc_ctr(coef_rep, xs_sc, ys_sc)
    return jnp.concatenate([out_sc, out_tc.reshape(LTCPAD)[:LTC]])
